# Initial kernel scaffold; baseline (speedup 1.0000x reference)
import jax
import jax.numpy as jnp
from jax.experimental import pallas as pl


def kernel(numbers, edge_index, batch, emb_table, W_msg, W_self, W_nbr, b):
    raise NotImplementedError("write your pallas kernel here")


# speedup vs baseline: 22.3599x; 22.3599x over previous
import jax
import jax.numpy as jnp
from jax import lax
from jax.experimental import pallas as pl
from jax.experimental.pallas import tpu as pltpu
from jax.experimental.pallas import tpu_sc as plsc

N = 10000
E = 320000
D = 128
G = 64
V = 100

NPAD = 10240
VP = 128
BN = 1024
NB = NPAD // BN
NTILES = 32
EPAD = 327680
EPT = EPAD // NTILES
CH = 5120
NCHUNK = EPT // CH
ROWSC = CH // 128
CHUNK = NPAD * VP // 16


def _sc_histogram_body(numbers_hbm, src_hbm, dst_hbm, zeros_hbm,
                       out_hbm, num_v, src_v, dst_v, idx_v, val_v, c_sp, sem):
    c = lax.axis_index("c")
    s = lax.axis_index("s")
    wid = s * 2 + c
    off = s * CHUNK

    pltpu.sync_copy(zeros_hbm.at[pl.ds(off, CHUNK)], c_sp.at[pl.ds(off, CHUNK)])

    pltpu.sync_copy(numbers_hbm, num_v)

    plsc.subcore_barrier()

    lane = lax.iota(jnp.int32, 16)
    one = jnp.full((16,), 1.0, jnp.float32)
    zero = jnp.zeros((16,), jnp.float32)

    for q in range(NCHUNK):
        ebase = wid * EPT + q * CH
        pltpu.sync_copy(src_hbm.at[pl.ds(ebase, CH)], src_v)
        pltpu.sync_copy(dst_hbm.at[pl.ds(ebase, CH)], dst_v)

        def row(j, carry):
            for cc in range(8):
                base = j * 128 + cc * 16
                s16 = src_v[pl.ds(base, 16)]
                d16 = dst_v[pl.ds(base, 16)]
                k16 = plsc.load_gather(num_v, [s16])
                idx_v[j, pl.ds(cc * 16, 16)] = d16 * VP + k16
                gpos = ebase + base + lane
                val_v[j, pl.ds(cc * 16, 16)] = jnp.where(gpos < E, one, zero)
            return carry

        lax.fori_loop(0, ROWSC, row, 0)

        def scatter_step(t, carry):
            descs = []
            for k in range(10):
                j = t * 10 + k
                descs.append(pltpu.async_copy(
                    val_v.at[j], c_sp.at[idx_v.at[j]], sem, add=True))
            for dsc in descs:
                dsc.wait()
            return carry

        lax.fori_loop(0, ROWSC // 10, scatter_step, 0)

    plsc.subcore_barrier()

    pltpu.sync_copy(c_sp.at[pl.ds(off, CHUNK)], out_hbm.at[c, pl.ds(off, CHUNK)])


def _make_sc_histogram():
    return pl.kernel(
        _sc_histogram_body,
        out_type=jax.ShapeDtypeStruct((2, NPAD * VP), jnp.float32),
        mesh=plsc.VectorSubcoreMesh(core_axis_name="c", subcore_axis_name="s"),
        compiler_params=pltpu.CompilerParams(needs_layout_passes=False),
        scratch_types=[
            pltpu.VMEM((NPAD,), jnp.int32),
            pltpu.VMEM((CH,), jnp.int32),
            pltpu.VMEM((CH,), jnp.int32),
            pltpu.VMEM((ROWSC, 128), jnp.int32),
            pltpu.VMEM((ROWSC, 128), jnp.float32),
            pltpu.VMEM_SHARED((NPAD * VP,), jnp.float32),
            pltpu.SemaphoreType.DMA,
        ],
    )


def _tc_body(cpair_ref, num_ref, bat_ref, emb_ref, wm_ref, ws_ref, wn_ref,
             b_ref, out_ref, acc_sum, acc_cnt):
    i = pl.program_id(0)
    prec = lax.Precision.HIGHEST
    dot = lambda a, b, da, db: lax.dot_general(
        a, b, (((da,), (db,)), ((), ())), precision=prec)

    e_self = dot(emb_ref[...], ws_ref[...], 1, 0)
    w_cmb = dot(wm_ref[...], wn_ref[...], 1, 0)
    e_mn = dot(emb_ref[...], w_cmb, 1, 0)

    cb = cpair_ref[0] + cpair_ref[1]
    deg = jnp.sum(cb, axis=1, keepdims=True)
    p = cb / jnp.maximum(deg, 1.0)
    agg = dot(p, e_mn, 1, 0)

    nums = num_ref[0]
    vio = lax.broadcasted_iota(jnp.int32, (VP, 1), 0)
    onehot_t = (vio == nums).astype(jnp.float32)
    xs = dot(onehot_t, e_self, 0, 0)

    h = jnp.maximum(xs + agg + b_ref[0:1, :], 0.0)

    bat = bat_ref[0]
    gio = lax.broadcasted_iota(jnp.int32, (G, 1), 0)
    oh_g = (gio == bat).astype(jnp.float32)
    sums = dot(oh_g, h, 1, 0)
    cnts = jnp.sum(oh_g, axis=1, keepdims=True)

    @pl.when(i == 0)
    def _init():
        acc_sum[...] = jnp.zeros_like(acc_sum)
        acc_cnt[...] = jnp.zeros_like(acc_cnt)

    acc_sum[...] += sums
    acc_cnt[...] += jnp.broadcast_to(cnts, (G, D))

    @pl.when(i == NB - 1)
    def _fin():
        out_ref[...] = acc_sum[...] / jnp.maximum(acc_cnt[...], 1.0)


def _tc_readout(cpair, numbers3, batch3, emb_p, wm, ws, wn, b2):
    return pl.pallas_call(
        _tc_body,
        grid=(NB,),
        in_specs=[
            pl.BlockSpec((2, BN, VP), lambda i: (0, i, 0)),
            pl.BlockSpec((1, 1, BN), lambda i: (i, 0, 0)),
            pl.BlockSpec((1, 1, BN), lambda i: (i, 0, 0)),
            pl.BlockSpec((VP, D), lambda i: (0, 0)),
            pl.BlockSpec((D, D), lambda i: (0, 0)),
            pl.BlockSpec((D, D), lambda i: (0, 0)),
            pl.BlockSpec((D, D), lambda i: (0, 0)),
            pl.BlockSpec((8, D), lambda i: (0, 0)),
        ],
        out_specs=pl.BlockSpec((G, D), lambda i: (0, 0)),
        out_shape=jax.ShapeDtypeStruct((G, D), jnp.float32),
        scratch_shapes=[pltpu.VMEM((G, D), jnp.float32),
                        pltpu.VMEM((G, D), jnp.float32)],
    )(cpair, numbers3, batch3, emb_p, wm, ws, wn, b2)


def kernel(numbers, edge_index, batch, emb_table, W_msg, W_self, W_nbr, b):
    src = edge_index[0]
    dst = edge_index[1]
    numbers_p = jnp.pad(numbers, (0, NPAD - N))
    src_p = jnp.pad(src, (0, EPAD - E))
    dst_p = jnp.pad(dst, (0, EPAD - E))
    zeros = jnp.zeros((NPAD * VP,), jnp.float32)

    cpair = _make_sc_histogram()(numbers_p, src_p, dst_p, zeros)
    cpair = cpair.reshape(2, NPAD, VP)

    numbers3 = numbers_p.reshape(NB, 1, BN)
    batch3 = jnp.pad(batch, (0, NPAD - N),
                     constant_values=G).reshape(NB, 1, BN)
    emb_p = jnp.pad(emb_table, ((0, VP - V), (0, 0)))
    b2 = jnp.broadcast_to(b[None, :], (8, D))
    return _tc_readout(cpair, numbers3, batch3, emb_p, W_msg, W_self, W_nbr, b2)


# speedup vs baseline: 27.0367x; 1.2092x over previous
import jax
import jax.numpy as jnp
from jax import lax
from jax.experimental import pallas as pl
from jax.experimental.pallas import tpu as pltpu
from jax.experimental.pallas import tpu_sc as plsc

N = 10000
E = 320000
D = 128
G = 64
V = 100

NPAD = 10240
VP = 128
BN = 2048
NB = NPAD // BN
NTILES = 32
EPAD = 327680
EPT = EPAD // NTILES
CH = 5120
NCHUNK = EPT // CH
ROWSC = CH // 128
CHUNK = NPAD * VP // 16


def _sc_histogram_body(numbers_hbm, src_hbm, dst_hbm,
                       out_hbm, num_v, src_v, dst_v, idx_v, val_v, c_sp, sem):
    c = lax.axis_index("c")
    s = lax.axis_index("s")
    wid = s * 2 + c
    off = s * CHUNK

    lane = lax.iota(jnp.int32, 16)
    one = jnp.full((16,), 1.0, jnp.float32)
    zero = jnp.zeros((16,), jnp.float32)

    def zrow(i, carry):
        val_v[pl.ds(i * 16, 16)] = zero
        return carry

    lax.fori_loop(0, CH // 16, zrow, 0)

    def zcopy(k, carry):
        pltpu.sync_copy(val_v, c_sp.at[pl.ds(off + k * CH, CH)])
        return carry

    lax.fori_loop(0, CHUNK // CH, zcopy, 0)

    pltpu.sync_copy(numbers_hbm, num_v)

    plsc.subcore_barrier()

    for q in range(NCHUNK):
        ebase = wid * EPT + q * CH
        pltpu.sync_copy(src_hbm.at[pl.ds(ebase, CH)], src_v)
        pltpu.sync_copy(dst_hbm.at[pl.ds(ebase, CH)], dst_v)

        def row(j, carry):
            for cc in range(8):
                base = j * 128 + cc * 16
                s16 = src_v[pl.ds(base, 16)]
                d16 = dst_v[pl.ds(base, 16)]
                k16 = plsc.load_gather(num_v, [s16])
                idx_v[pl.ds(base, 16)] = d16 * VP + k16
                gpos = ebase + base + lane
                val_v[pl.ds(base, 16)] = jnp.where(gpos < E, one, zero)
            return carry

        lax.fori_loop(0, CH // 128, row, 0)

        pltpu.sync_copy(val_v, c_sp.at[idx_v], add=True)

    plsc.subcore_barrier()

    pltpu.sync_copy(c_sp.at[pl.ds(off, CHUNK)], out_hbm.at[c, pl.ds(off, CHUNK)])


def _make_sc_histogram():
    return pl.kernel(
        _sc_histogram_body,
        out_type=jax.ShapeDtypeStruct((2, NPAD * VP), jnp.float32),
        mesh=plsc.VectorSubcoreMesh(core_axis_name="c", subcore_axis_name="s"),
        compiler_params=pltpu.CompilerParams(needs_layout_passes=False),
        scratch_types=[
            pltpu.VMEM((NPAD,), jnp.int32),
            pltpu.VMEM((CH,), jnp.int32),
            pltpu.VMEM((CH,), jnp.int32),
            pltpu.VMEM((CH,), jnp.int32),
            pltpu.VMEM((CH,), jnp.float32),
            pltpu.VMEM_SHARED((NPAD * VP,), jnp.float32),
            pltpu.SemaphoreType.DMA,
        ],
    )


def _tc_body(cpair_ref, num_ref, bat_ref, emb_ref, wm_ref, ws_ref, wn_ref,
             b_ref, out_ref, acc_sum, acc_cnt, e_self_s, e_mn_s):
    i = pl.program_id(0)
    prec = lax.Precision.HIGHEST
    dot = lambda a, b, da, db, p=prec: lax.dot_general(
        a, b, (((da,), (db,)), ((), ())), precision=p)

    @pl.when(i == 0)
    def _tables():
        e_self_s[...] = dot(emb_ref[...], ws_ref[...], 1, 0)
        w_cmb = dot(wm_ref[...], wn_ref[...], 1, 0)
        e_mn_s[...] = dot(emb_ref[...], w_cmb, 1, 0)

    e_self = e_self_s[...]
    e_mn = e_mn_s[...]

    cb = cpair_ref[0] + cpair_ref[1]
    deg = jnp.sum(cb, axis=1, keepdims=True)
    p = cb / jnp.maximum(deg, 1.0)
    hi = lax.Precision.DEFAULT
    agg = dot(p, e_mn, 1, 0, hi)

    nums = num_ref[0]
    vio = lax.broadcasted_iota(jnp.int32, (VP, 1), 0)
    onehot_t = (vio == nums).astype(jnp.float32)
    xs = dot(onehot_t, e_self, 0, 0, hi)

    h = jnp.maximum(xs + agg + b_ref[0:1, :], 0.0)

    bat = bat_ref[0]
    gio = lax.broadcasted_iota(jnp.int32, (G, 1), 0)
    oh_g = (gio == bat).astype(jnp.float32)
    sums = dot(oh_g, h, 1, 0, hi)
    cnts = jnp.sum(oh_g, axis=1, keepdims=True)

    @pl.when(i == 0)
    def _init():
        acc_sum[...] = jnp.zeros_like(acc_sum)
        acc_cnt[...] = jnp.zeros_like(acc_cnt)

    acc_sum[...] += sums
    acc_cnt[...] += jnp.broadcast_to(cnts, (G, D))

    @pl.when(i == NB - 1)
    def _fin():
        out_ref[...] = acc_sum[...] / jnp.maximum(acc_cnt[...], 1.0)


def _tc_readout(cpair, numbers3, batch3, emb_p, wm, ws, wn, b2):
    return pl.pallas_call(
        _tc_body,
        grid=(NB,),
        in_specs=[
            pl.BlockSpec((2, BN, VP), lambda i: (0, i, 0)),
            pl.BlockSpec((1, 1, BN), lambda i: (i, 0, 0)),
            pl.BlockSpec((1, 1, BN), lambda i: (i, 0, 0)),
            pl.BlockSpec((VP, D), lambda i: (0, 0)),
            pl.BlockSpec((D, D), lambda i: (0, 0)),
            pl.BlockSpec((D, D), lambda i: (0, 0)),
            pl.BlockSpec((D, D), lambda i: (0, 0)),
            pl.BlockSpec((8, D), lambda i: (0, 0)),
        ],
        out_specs=pl.BlockSpec((G, D), lambda i: (0, 0)),
        out_shape=jax.ShapeDtypeStruct((G, D), jnp.float32),
        scratch_shapes=[pltpu.VMEM((G, D), jnp.float32),
                        pltpu.VMEM((G, D), jnp.float32),
                        pltpu.VMEM((VP, D), jnp.float32),
                        pltpu.VMEM((VP, D), jnp.float32)],
    )(cpair, numbers3, batch3, emb_p, wm, ws, wn, b2)


def kernel(numbers, edge_index, batch, emb_table, W_msg, W_self, W_nbr, b):
    src = edge_index[0]
    dst = edge_index[1]
    numbers_p = jnp.pad(numbers, (0, NPAD - N))
    src_p = jnp.pad(src, (0, EPAD - E))
    dst_p = jnp.pad(dst, (0, EPAD - E))

    cpair = _make_sc_histogram()(numbers_p, src_p, dst_p)
    cpair = cpair.reshape(2, NPAD, VP)

    numbers3 = numbers_p.reshape(NB, 1, BN)
    batch3 = jnp.pad(batch, (0, NPAD - N),
                     constant_values=G).reshape(NB, 1, BN)
    emb_p = jnp.pad(emb_table, ((0, VP - V), (0, 0)))
    b2 = jnp.broadcast_to(b[None, :], (8, D))
    return _tc_readout(cpair, numbers3, batch3, emb_p, W_msg, W_self, W_nbr, b2)


# speedup vs baseline: 36.9545x; 1.3668x over previous
import jax
import jax.numpy as jnp
from jax import lax
from jax.experimental import pallas as pl
from jax.experimental.pallas import tpu as pltpu
from jax.experimental.pallas import tpu_sc as plsc

N = 10000
E = 320000
D = 128
G = 64
V = 100

NPAD = 10240
VP = 128
BN = 2048
NB = NPAD // BN
NTILES = 32
EPAD = 327680
EPT = EPAD // NTILES
CH = 5120
NCHUNK = EPT // CH
ROWSC = CH // 128
CHUNK = NPAD * VP // 16


def _sc_histogram_body(numbers_hbm, ei_hbm, out_hbm,
                       src_v, dst_v, k_v, idx_v, val_v, zbuf,
                       num_sp, c_sp, sem, semz):
    c = lax.axis_index("c")
    s = lax.axis_index("s")
    wid = s * 2 + c
    off = s * CHUNK

    lane = lax.iota(jnp.int32, 16)
    one = jnp.full((16,), 1.0, jnp.float32)
    zero = jnp.zeros((16,), jnp.float32)

    def zrow(i, carry):
        zbuf[pl.ds(i * 16, 16)] = zero
        return carry

    lax.fori_loop(0, CH // 16, zrow, 0)
    zcopies = [pltpu.async_copy(zbuf, c_sp.at[pl.ds(off + k * CH, CH)], semz)
               for k in range(CHUNK // CH)]

    nslice = NPAD // 16
    pltpu.sync_copy(numbers_hbm.at[pl.ds(s * nslice, nslice)],
                    num_sp.at[pl.ds(s * nslice, nslice)])

    def orow(i, carry):
        val_v[pl.ds(i * 16, 16)] = one
        return carry

    lax.fori_loop(0, CH // 16, orow, 0)

    for zc in zcopies:
        zc.wait()
    plsc.subcore_barrier()

    for q in range(NCHUNK):
        ebase = wid * EPT + q * CH
        pltpu.sync_copy(ei_hbm.at[0, pl.ds(ebase, CH)], src_v)
        pltpu.sync_copy(ei_hbm.at[1, pl.ds(ebase, CH)], dst_v)

        pltpu.async_copy(num_sp.at[src_v], k_v, sem).wait()

        def row(j, carry):
            b = j * 16
            d16 = dst_v[pl.ds(b, 16)]
            k16 = k_v[pl.ds(b, 16)]
            idx_v[pl.ds(b, 16)] = d16 * VP + k16
            return carry

        lax.fori_loop(0, CH // 16, row, 0)

        lb = jnp.clip((E - ebase) // 16, 0, CH // 16)

        def vfix(t, carry):
            gpos = ebase + t * 16 + lane
            val_v[pl.ds(t * 16, 16)] = jnp.where(gpos < E, one, zero)
            return carry

        lax.fori_loop(lb, CH // 16, vfix, 0)

        pltpu.sync_copy(val_v, c_sp.at[idx_v], add=True)

    plsc.subcore_barrier()

    pltpu.sync_copy(c_sp.at[pl.ds(off, CHUNK)],
                    out_hbm.at[pl.ds(c * (NPAD * VP) + off, CHUNK)])


def _make_sc_histogram():
    return pl.kernel(
        _sc_histogram_body,
        out_type=jax.ShapeDtypeStruct((2 * NPAD * VP,), jnp.float32),
        mesh=plsc.VectorSubcoreMesh(core_axis_name="c", subcore_axis_name="s"),
        compiler_params=pltpu.CompilerParams(needs_layout_passes=False),
        scratch_types=[
            pltpu.VMEM((CH,), jnp.int32),
            pltpu.VMEM((CH,), jnp.int32),
            pltpu.VMEM((CH,), jnp.int32),
            pltpu.VMEM((CH,), jnp.int32),
            pltpu.VMEM((CH,), jnp.float32),
            pltpu.VMEM((CH,), jnp.float32),
            pltpu.VMEM_SHARED((NPAD,), jnp.int32),
            pltpu.VMEM_SHARED((NPAD * VP,), jnp.float32),
            pltpu.SemaphoreType.DMA,
            pltpu.SemaphoreType.DMA,
        ],
    )


def _tc_body(cpair_ref, num_ref, bat_ref, emb_ref, wm_ref, ws_ref, wn_ref,
             b_ref, out_ref, acc_sum, acc_cnt, e_self_s, e_mn_s):
    i = pl.program_id(0)
    prec = lax.Precision.HIGHEST
    dot = lambda a, b, da, db, p=prec: lax.dot_general(
        a, b, (((da,), (db,)), ((), ())), precision=p)

    @pl.when(i == 0)
    def _tables():
        e_self_s[...] = dot(emb_ref[...], ws_ref[...], 1, 0)
        w_cmb = dot(wm_ref[...], wn_ref[...], 1, 0)
        e_mn_s[...] = dot(emb_ref[...], w_cmb, 1, 0)

    e_self = e_self_s[...]
    e_mn = e_mn_s[...]

    cb = cpair_ref[0] + cpair_ref[1]
    deg = jnp.sum(cb, axis=1, keepdims=True)
    p = cb / jnp.maximum(deg, 1.0)
    hi = lax.Precision.DEFAULT
    agg = dot(p, e_mn, 1, 0, hi)

    nums = num_ref[0]
    vio = lax.broadcasted_iota(jnp.int32, (VP, 1), 0)
    onehot_t = (vio == nums).astype(jnp.float32)
    xs = dot(onehot_t, e_self, 0, 0, hi)

    h = jnp.maximum(xs + agg + b_ref[0:1, :], 0.0)

    bat = bat_ref[0]
    gio = lax.broadcasted_iota(jnp.int32, (G, 1), 0)
    oh_g = (gio == bat).astype(jnp.float32)
    sums = dot(oh_g, h, 1, 0, hi)
    cnts = jnp.sum(oh_g, axis=1, keepdims=True)

    @pl.when(i == 0)
    def _init():
        acc_sum[...] = jnp.zeros_like(acc_sum)
        acc_cnt[...] = jnp.zeros_like(acc_cnt)

    acc_sum[...] += sums
    acc_cnt[...] += jnp.broadcast_to(cnts, (G, D))

    @pl.when(i == NB - 1)
    def _fin():
        out_ref[...] = acc_sum[...] / jnp.maximum(acc_cnt[...], 1.0)


def _tc_readout(cpair, numbers3, batch3, emb_p, wm, ws, wn, b2):
    return pl.pallas_call(
        _tc_body,
        grid=(NB,),
        in_specs=[
            pl.BlockSpec((2, BN, VP), lambda i: (0, i, 0)),
            pl.BlockSpec((1, 1, BN), lambda i: (i, 0, 0)),
            pl.BlockSpec((1, 1, BN), lambda i: (i, 0, 0)),
            pl.BlockSpec((VP, D), lambda i: (0, 0)),
            pl.BlockSpec((D, D), lambda i: (0, 0)),
            pl.BlockSpec((D, D), lambda i: (0, 0)),
            pl.BlockSpec((D, D), lambda i: (0, 0)),
            pl.BlockSpec((8, D), lambda i: (0, 0)),
        ],
        out_specs=pl.BlockSpec((G, D), lambda i: (0, 0)),
        out_shape=jax.ShapeDtypeStruct((G, D), jnp.float32),
        scratch_shapes=[pltpu.VMEM((G, D), jnp.float32),
                        pltpu.VMEM((G, D), jnp.float32),
                        pltpu.VMEM((VP, D), jnp.float32),
                        pltpu.VMEM((VP, D), jnp.float32)],
    )(cpair, numbers3, batch3, emb_p, wm, ws, wn, b2)


def kernel(numbers, edge_index, batch, emb_table, W_msg, W_self, W_nbr, b):
    numbers_p = jnp.pad(numbers, (0, NPAD - N))
    ei_p = jnp.pad(edge_index, ((0, 0), (0, EPAD - E)))

    cflat = _make_sc_histogram()(numbers_p, ei_p)
    cpair = cflat.reshape(2, NPAD, VP)

    numbers3 = numbers_p.reshape(NB, 1, BN)
    batch3 = jnp.pad(batch, (0, NPAD - N),
                     constant_values=G).reshape(NB, 1, BN)
    emb_p = jnp.pad(emb_table, ((0, VP - V), (0, 0)))
    b2 = jnp.broadcast_to(b[None, :], (8, D))
    return _tc_readout(cpair, numbers3, batch3, emb_p, W_msg, W_self, W_nbr, b2)
